# Initial kernel scaffold; baseline (speedup 1.0000x reference)
import jax
import jax.numpy as jnp
from jax.experimental import pallas as pl


def kernel(s_feat_map, t_feat_map, s_label_pixel, t_probs_pixel):
    raise NotImplementedError("write your pallas kernel here")


# speedup vs baseline: 1.9203x; 1.9203x over previous
import jax
import jax.numpy as jnp
import numpy as np
from jax.experimental import pallas as pl
from jax.experimental.pallas import tpu as pltpu

_NUM_CLASSES = 19
_PURITY_T = 0.9
_CONF_T = 0.9
_IGNORE = -1
_MOM = 0.99

_B = 4
_C = 256
_HC = 32
_HF = 64
_N = _HF * _HF
_IMG = 512
_PP = (_IMG // _HF) ** 2
_RT = 8


def _up_matrix(n_in: int) -> np.ndarray:
    n_out = 2 * n_in
    a = np.zeros((n_out, n_in), np.float32)
    for i in range(n_out):
        c = (i + 0.5) / 2.0 - 0.5
        lo = int(np.floor(c))
        w = c - lo
        l0 = min(max(lo, 0), n_in - 1)
        l1 = min(max(lo + 1, 0), n_in - 1)
        a[i, l0] += 1.0 - w
        a[i, l1] += w
    return a


_A = _up_matrix(_HC)
_MT = np.kron(_A, _A).T.copy()


def _conf_lab_body(probs_ref, conf_ref, labf_ref):
    conf = probs_ref[0, 0]
    lab = jnp.zeros_like(conf, dtype=jnp.int32)
    for c in range(1, _NUM_CLASSES):
        p = probs_ref[0, c]
        upd = p > conf
        conf = jnp.where(upd, p, conf)
        lab = jnp.where(upd, c, lab)
    conf_ref[0] = conf
    labf_ref[0] = jnp.where(conf < _CONF_T, _IGNORE, lab)


def _main_body(mt_ref, s_feat_ref, t_feat_ref, sp_ref, tp_ref, cp_ref,
               s_tokt_ref, t_tokt_ref, s_proto_ref, t_proto_ref,
               s_d_ref, t_d_ref):
    b = pl.program_id(0)

    @pl.when(b == 0)
    def _init():
        s_proto_ref[...] = jnp.zeros_like(s_proto_ref)
        t_proto_ref[...] = jnp.zeros_like(t_proto_ref)

    mt = mt_ref[...]
    s_tokt = jnp.dot(s_feat_ref[0], mt, preferred_element_type=jnp.float32)
    t_tokt = jnp.dot(t_feat_ref[0], mt, preferred_element_type=jnp.float32)
    s_tokt_ref[0] = s_tokt
    t_tokt_ref[0] = t_tokt

    def stats(patches):
        cnt_ig = jnp.sum((patches == _IGNORE).astype(jnp.int32), axis=0,
                         keepdims=True)
        maxc = cnt_ig
        mode = jnp.full((1, _N), _IGNORE, jnp.int32)
        for v in range(_NUM_CLASSES):
            cnt = jnp.sum((patches == v).astype(jnp.int32), axis=0,
                          keepdims=True)
            upd = cnt > maxc
            maxc = jnp.where(upd, cnt, maxc)
            mode = jnp.where(upd, v, mode)
        nvalid = _PP - cnt_ig
        num = jnp.where(mode == _IGNORE, 0, maxc).astype(jnp.float32)
        purity = num / jnp.maximum(nvalid.astype(jnp.float32), 1.0)
        mask = (purity >= _PURITY_T) & (nvalid > 0)
        return mask, mode

    s_mask, s_mode = stats(sp_ref[0])
    t_mask, t_mode = stats(tp_ref[0])
    conf_tok = jnp.sum(cp_ref[0], axis=0, keepdims=True) * (1.0 / _PP)
    t_mask = t_mask & (conf_tok >= _CONF_T)

    iota = jax.lax.broadcasted_iota(jnp.int32, (_NUM_CLASSES, _N), 0)
    s_any = jnp.max(s_mask.astype(jnp.int32), axis=1, keepdims=True) > 0
    t_any = jnp.max(t_mask.astype(jnp.int32), axis=1, keepdims=True) > 0

    def proto_update(mask, mode, tokt, proto_ref, gate):
        oh = ((iota == mode) & mask).astype(jnp.float32)
        sums = jax.lax.dot_general(oh, tokt, (((1,), (1,)), ((), ())),
                                   preferred_element_type=jnp.float32)
        counts = jnp.sum(oh, axis=1, keepdims=True)
        means = sums / jnp.maximum(counts, 1.0)
        upd = (counts > 0) & gate
        prev = proto_ref[...]
        newp = jnp.where(upd, _MOM * prev + (1.0 - _MOM) * means, prev)
        proto_ref[...] = newp
        return newp

    s_newp = proto_update(s_mask, s_mode, s_tokt, s_proto_ref, s_any)
    t_newp = proto_update(t_mask, t_mode, t_tokt, t_proto_ref, t_any & s_any)

    def dists(mask, mode, tokt, newp):
        sel = (iota == jnp.clip(mode, 0, _NUM_CLASSES - 1)).astype(jnp.float32)
        z = jnp.dot(newp, tokt, preferred_element_type=jnp.float32)
        xp = jnp.sum(sel * z, axis=0, keepdims=True)
        x2 = jnp.sum(tokt * tokt, axis=0, keepdims=True)
        p2 = jnp.sum(newp * newp, axis=1, keepdims=True)
        p2t = jnp.sum(sel * p2, axis=0, keepdims=True)
        d2 = jnp.maximum(x2 - 2.0 * xp + p2t, 0.0)
        return jnp.sqrt(d2) * mask.astype(jnp.float32)

    s_d_ref[0] = dists(s_mask, s_mode, s_tokt, s_newp)
    t_d_ref[0] = dists(t_mask, t_mode, t_tokt, t_newp)


def kernel(s_feat_map, t_feat_map, s_label_pixel, t_probs_pixel):
    rows = _IMG // _RT

    conf, labf = pl.pallas_call(
        _conf_lab_body,
        grid=(_B, _RT),
        in_specs=[pl.BlockSpec((1, _NUM_CLASSES, rows, _IMG),
                               lambda b, r: (b, 0, r, 0))],
        out_specs=[pl.BlockSpec((1, rows, _IMG), lambda b, r: (b, r, 0)),
                   pl.BlockSpec((1, rows, _IMG), lambda b, r: (b, r, 0))],
        out_shape=[jax.ShapeDtypeStruct((_B, _IMG, _IMG), jnp.float32),
                   jax.ShapeDtypeStruct((_B, _IMG, _IMG), jnp.int32)],
        compiler_params=pltpu.CompilerParams(
            dimension_semantics=("parallel", "arbitrary")),
    )(t_probs_pixel)

    def patchify(x):
        return (x.reshape(_B, _HF, 8, _HF, 8)
                 .transpose(0, 2, 4, 1, 3)
                 .reshape(_B, _PP, _N))

    sp = patchify(s_label_pixel.astype(jnp.int32))
    tp = patchify(labf)
    cp = patchify(conf)
    s_feat = s_feat_map.reshape(_B, _C, _HC * _HC)
    t_feat = t_feat_map.reshape(_B, _C, _HC * _HC)
    mt = jnp.asarray(_MT)

    s_tokt, t_tokt, s_proto, t_proto, s_d, t_d = pl.pallas_call(
        _main_body,
        grid=(_B,),
        in_specs=[
            pl.BlockSpec((_HC * _HC, _N), lambda b: (0, 0)),
            pl.BlockSpec((1, _C, _HC * _HC), lambda b: (b, 0, 0)),
            pl.BlockSpec((1, _C, _HC * _HC), lambda b: (b, 0, 0)),
            pl.BlockSpec((1, _PP, _N), lambda b: (b, 0, 0)),
            pl.BlockSpec((1, _PP, _N), lambda b: (b, 0, 0)),
            pl.BlockSpec((1, _PP, _N), lambda b: (b, 0, 0)),
        ],
        out_specs=[
            pl.BlockSpec((1, _C, _N), lambda b: (b, 0, 0)),
            pl.BlockSpec((1, _C, _N), lambda b: (b, 0, 0)),
            pl.BlockSpec((_NUM_CLASSES, _C), lambda b: (0, 0)),
            pl.BlockSpec((_NUM_CLASSES, _C), lambda b: (0, 0)),
            pl.BlockSpec((1, 1, _N), lambda b: (b, 0, 0)),
            pl.BlockSpec((1, 1, _N), lambda b: (b, 0, 0)),
        ],
        out_shape=[
            jax.ShapeDtypeStruct((_B, _C, _N), jnp.float32),
            jax.ShapeDtypeStruct((_B, _C, _N), jnp.float32),
            jax.ShapeDtypeStruct((_NUM_CLASSES, _C), jnp.float32),
            jax.ShapeDtypeStruct((_NUM_CLASSES, _C), jnp.float32),
            jax.ShapeDtypeStruct((_B, 1, _N), jnp.float32),
            jax.ShapeDtypeStruct((_B, 1, _N), jnp.float32),
        ],
    )(mt, s_feat, t_feat, sp, tp, cp)

    s_tok = s_tokt.transpose(0, 2, 1)
    t_tok = t_tokt.transpose(0, 2, 1)
    return (s_tok, t_tok, s_proto, t_proto,
            s_d.reshape(_B, _N), t_d.reshape(_B, _N))


# speedup vs baseline: 3.0551x; 1.5910x over previous
import jax
import jax.numpy as jnp
import numpy as np
from jax.experimental import pallas as pl
from jax.experimental.pallas import tpu as pltpu

_NUM_CLASSES = 19
_PURITY_T = 0.9
_CONF_T = 0.9
_IGNORE = -1
_MOM = 0.99

_B = 4
_C = 256
_HC = 32
_HF = 64
_N = _HF * _HF
_IMG = 512
_PH = _IMG // _HF
_PP = _PH * _PH
_RT = 8
_ROWS = _IMG // _RT
_NV = _NUM_CLASSES + 1


def _up_matrix(n_in: int) -> np.ndarray:
    n_out = 2 * n_in
    a = np.zeros((n_out, n_in), np.float32)
    for i in range(n_out):
        c = (i + 0.5) / 2.0 - 0.5
        lo = int(np.floor(c))
        w = c - lo
        l0 = min(max(lo, 0), n_in - 1)
        l1 = min(max(lo + 1, 0), n_in - 1)
        a[i, l0] += 1.0 - w
        a[i, l1] += w
    return a


_A = _up_matrix(_HC)
_MT = np.kron(_A, _A).T.copy()

_COLS = (np.arange(_IMG)[:, None] // _PH ==
         np.arange(_IMG // _PH)[None, :]).astype(np.float32)
_ROWSUM = (np.arange(_ROWS // _PH)[:, None] ==
           np.arange(_ROWS)[None, :] // _PH).astype(np.float32)


def _stats_body(cols_ref, rowsum_ref, probs_ref, slab_ref,
                tcnt_ref, scnt_ref, csum_ref):
    cols = cols_ref[...]
    rowsum = rowsum_ref[...]

    conf = probs_ref[0, 0]
    lab = jnp.zeros_like(conf, dtype=jnp.int32)
    for c in range(1, _NUM_CLASSES):
        p = probs_ref[0, c]
        upd = p > conf
        conf = jnp.where(upd, p, conf)
        lab = jnp.where(upd, c, lab)
    labf = jnp.where(conf < _CONF_T, _IGNORE, lab)
    slab = slab_ref[0]

    def pool(x):
        return jnp.dot(rowsum, jnp.dot(x, cols,
                                       preferred_element_type=jnp.float32),
                       preferred_element_type=jnp.float32)

    tcnt_ref[0, 0] = pool((labf == _IGNORE).astype(jnp.float32))
    scnt_ref[0, 0] = jnp.zeros((_RT, _HF), jnp.float32)
    for v in range(_NUM_CLASSES):
        tcnt_ref[0, v + 1] = pool((labf == v).astype(jnp.float32))
        scnt_ref[0, v + 1] = pool((slab == v).astype(jnp.float32))
    csum_ref[0] = pool(conf) * (1.0 / _PP)


def _main_body(mt_ref, s_feat_ref, t_feat_ref, scnt_ref, tcnt_ref, cp_ref,
               s_tok_ref, t_tok_ref, s_proto_ref, t_proto_ref,
               s_d_ref, t_d_ref):
    b = pl.program_id(0)

    @pl.when(b == 0)
    def _init():
        s_proto_ref[...] = jnp.zeros_like(s_proto_ref)
        t_proto_ref[...] = jnp.zeros_like(t_proto_ref)

    mt = mt_ref[...]
    s_tokt = jnp.dot(s_feat_ref[0], mt, preferred_element_type=jnp.float32)
    t_tokt = jnp.dot(t_feat_ref[0], mt, preferred_element_type=jnp.float32)
    s_tok_ref[0] = s_tokt.T
    t_tok_ref[0] = t_tokt.T

    def stats(cnt):
        cnt_ig = cnt[0:1, :]
        maxc = cnt_ig
        mode = jnp.full((1, _N), _IGNORE, jnp.int32)
        for v in range(_NUM_CLASSES):
            cv = cnt[v + 1:v + 2, :]
            upd = cv > maxc
            maxc = jnp.where(upd, cv, maxc)
            mode = jnp.where(upd, v, mode)
        nvalid = _PP - cnt_ig
        num = jnp.where(mode == _IGNORE, 0.0, maxc)
        purity = num / jnp.maximum(nvalid, 1.0)
        mask = (purity >= _PURITY_T) & (nvalid > 0)
        return mask, mode

    s_mask, s_mode = stats(scnt_ref[0])
    t_mask, t_mode = stats(tcnt_ref[0])
    conf_tok = cp_ref[0]
    t_mask = t_mask & (conf_tok >= _CONF_T)

    iota = jax.lax.broadcasted_iota(jnp.int32, (_NUM_CLASSES, _N), 0)
    s_any = jnp.max(s_mask.astype(jnp.int32), axis=1, keepdims=True) > 0
    t_any = jnp.max(t_mask.astype(jnp.int32), axis=1, keepdims=True) > 0

    def proto_update(mask, mode, tokt, proto_ref, gate):
        oh = ((iota == mode) & mask).astype(jnp.float32)
        sums = jax.lax.dot_general(oh, tokt, (((1,), (1,)), ((), ())),
                                   preferred_element_type=jnp.float32)
        counts = jnp.sum(oh, axis=1, keepdims=True)
        means = sums / jnp.maximum(counts, 1.0)
        upd = (counts > 0) & gate
        prev = proto_ref[...]
        newp = jnp.where(upd, _MOM * prev + (1.0 - _MOM) * means, prev)
        proto_ref[...] = newp
        return newp

    s_newp = proto_update(s_mask, s_mode, s_tokt, s_proto_ref, s_any)
    t_newp = proto_update(t_mask, t_mode, t_tokt, t_proto_ref, t_any & s_any)

    def dists(mask, mode, tokt, newp):
        sel = (iota == jnp.clip(mode, 0, _NUM_CLASSES - 1)).astype(jnp.float32)
        z = jnp.dot(newp, tokt, preferred_element_type=jnp.float32)
        xp = jnp.sum(sel * z, axis=0, keepdims=True)
        x2 = jnp.sum(tokt * tokt, axis=0, keepdims=True)
        p2 = jnp.sum(newp * newp, axis=1, keepdims=True)
        p2t = jnp.sum(sel * p2, axis=0, keepdims=True)
        d2 = jnp.maximum(x2 - 2.0 * xp + p2t, 0.0)
        return jnp.sqrt(d2) * mask.astype(jnp.float32)

    s_d_ref[0] = dists(s_mask, s_mode, s_tokt, s_newp)
    t_d_ref[0] = dists(t_mask, t_mode, t_tokt, t_newp)


def kernel(s_feat_map, t_feat_map, s_label_pixel, t_probs_pixel):
    cols = jnp.asarray(_COLS)
    rowsum = jnp.asarray(_ROWSUM)

    tcnt, scnt, csum = pl.pallas_call(
        _stats_body,
        grid=(_B, _RT),
        in_specs=[
            pl.BlockSpec((_IMG, _HF), lambda b, r: (0, 0)),
            pl.BlockSpec((_RT, _ROWS), lambda b, r: (0, 0)),
            pl.BlockSpec((1, _NUM_CLASSES, _ROWS, _IMG),
                         lambda b, r: (b, 0, r, 0)),
            pl.BlockSpec((1, _ROWS, _IMG), lambda b, r: (b, r, 0)),
        ],
        out_specs=[
            pl.BlockSpec((1, _NV, _RT, _HF), lambda b, r: (b, 0, r, 0)),
            pl.BlockSpec((1, _NV, _RT, _HF), lambda b, r: (b, 0, r, 0)),
            pl.BlockSpec((1, _RT, _HF), lambda b, r: (b, r, 0)),
        ],
        out_shape=[
            jax.ShapeDtypeStruct((_B, _NV, _HF, _HF), jnp.float32),
            jax.ShapeDtypeStruct((_B, _NV, _HF, _HF), jnp.float32),
            jax.ShapeDtypeStruct((_B, _HF, _HF), jnp.float32),
        ],
        compiler_params=pltpu.CompilerParams(
            dimension_semantics=("parallel", "arbitrary")),
    )(cols, rowsum, t_probs_pixel, s_label_pixel.astype(jnp.int32))

    scnt = scnt.reshape(_B, _NV, _N)
    tcnt = tcnt.reshape(_B, _NV, _N)
    csum = csum.reshape(_B, 1, _N)
    s_feat = s_feat_map.reshape(_B, _C, _HC * _HC)
    t_feat = t_feat_map.reshape(_B, _C, _HC * _HC)
    mt = jnp.asarray(_MT)

    s_tok, t_tok, s_proto, t_proto, s_d, t_d = pl.pallas_call(
        _main_body,
        grid=(_B,),
        in_specs=[
            pl.BlockSpec((_HC * _HC, _N), lambda b: (0, 0)),
            pl.BlockSpec((1, _C, _HC * _HC), lambda b: (b, 0, 0)),
            pl.BlockSpec((1, _C, _HC * _HC), lambda b: (b, 0, 0)),
            pl.BlockSpec((1, _NV, _N), lambda b: (b, 0, 0)),
            pl.BlockSpec((1, _NV, _N), lambda b: (b, 0, 0)),
            pl.BlockSpec((1, 1, _N), lambda b: (b, 0, 0)),
        ],
        out_specs=[
            pl.BlockSpec((1, _N, _C), lambda b: (b, 0, 0)),
            pl.BlockSpec((1, _N, _C), lambda b: (b, 0, 0)),
            pl.BlockSpec((_NUM_CLASSES, _C), lambda b: (0, 0)),
            pl.BlockSpec((_NUM_CLASSES, _C), lambda b: (0, 0)),
            pl.BlockSpec((1, 1, _N), lambda b: (b, 0, 0)),
            pl.BlockSpec((1, 1, _N), lambda b: (b, 0, 0)),
        ],
        out_shape=[
            jax.ShapeDtypeStruct((_B, _N, _C), jnp.float32),
            jax.ShapeDtypeStruct((_B, _N, _C), jnp.float32),
            jax.ShapeDtypeStruct((_NUM_CLASSES, _C), jnp.float32),
            jax.ShapeDtypeStruct((_NUM_CLASSES, _C), jnp.float32),
            jax.ShapeDtypeStruct((_B, 1, _N), jnp.float32),
            jax.ShapeDtypeStruct((_B, 1, _N), jnp.float32),
        ],
    )(mt, s_feat, t_feat, scnt, tcnt, csum)

    return (s_tok, t_tok, s_proto, t_proto,
            s_d.reshape(_B, _N), t_d.reshape(_B, _N))


# speedup vs baseline: 6.6321x; 2.1708x over previous
import jax
import jax.numpy as jnp
import numpy as np
from jax.experimental import pallas as pl
from jax.experimental.pallas import tpu as pltpu

_NUM_CLASSES = 19
_PURITY_T = 0.9
_CONF_T = 0.9
_IGNORE = -1
_MOM = 0.99

_B = 4
_C = 256
_HC = 32
_HF = 64
_N = _HF * _HF
_IMG = 512
_PH = _IMG // _HF
_PP = _PH * _PH
_RT = 8
_ROWS = _IMG // _RT
_NV = _NUM_CLASSES + 1


def _up_matrix(n_in: int) -> np.ndarray:
    n_out = 2 * n_in
    a = np.zeros((n_out, n_in), np.float32)
    for i in range(n_out):
        c = (i + 0.5) / 2.0 - 0.5
        lo = int(np.floor(c))
        w = c - lo
        l0 = min(max(lo, 0), n_in - 1)
        l1 = min(max(lo + 1, 0), n_in - 1)
        a[i, l0] += 1.0 - w
        a[i, l1] += w
    return a


_A = _up_matrix(_HC)
_MT = np.kron(_A, _A).T.copy()

_COLS = (np.arange(_IMG)[:, None] // _PH ==
         np.arange(_IMG // _PH)[None, :]).astype(np.float32)
_NBLK = _NV + _NUM_CLASSES + 1
_E_ROWS = _NBLK * _ROWS
_RS = (np.arange(_NBLK * _RT)[:, None] ==
       np.arange(_E_ROWS)[None, :] // _PH).astype(np.float32)


def _stats_body(cols_ref, rs_ref, probs_ref, slab_ref,
                tcnt_ref, scnt_ref, csum_ref):
    conf = probs_ref[0, 0]
    lab = jnp.zeros_like(conf, dtype=jnp.int32)
    for c in range(1, _NUM_CLASSES):
        p = probs_ref[0, c]
        upd = p > conf
        conf = jnp.where(upd, p, conf)
        lab = jnp.where(upd, c, lab)
    labf = jnp.where(conf < _CONF_T, _IGNORE, lab)
    slab = slab_ref[0]

    blocks = [(labf == (v - 1)).astype(jnp.float32) for v in range(_NV)]
    blocks += [(slab == v).astype(jnp.float32) for v in range(_NUM_CLASSES)]
    blocks += [conf]
    stack = jnp.concatenate(blocks, axis=0)
    pooled = jnp.dot(rs_ref[...],
                     jnp.dot(stack, cols_ref[...],
                             preferred_element_type=jnp.float32),
                     preferred_element_type=jnp.float32)

    for v in range(_NV):
        tcnt_ref[0, v] = pooled[v * _RT:(v + 1) * _RT]
    scnt_ref[0, 0] = jnp.zeros((_RT, _HF), jnp.float32)
    for v in range(_NUM_CLASSES):
        blk = _NV + v
        scnt_ref[0, v + 1] = pooled[blk * _RT:(blk + 1) * _RT]
    csum_ref[0] = pooled[(_NBLK - 1) * _RT:] * (1.0 / _PP)


def _main_body(mt_ref, s_feat_ref, t_feat_ref, scnt_ref, tcnt_ref, cp_ref,
               s_tok_ref, t_tok_ref, s_proto_ref, t_proto_ref,
               s_d_ref, t_d_ref):
    b = pl.program_id(0)

    @pl.when(b == 0)
    def _init():
        s_proto_ref[...] = jnp.zeros_like(s_proto_ref)
        t_proto_ref[...] = jnp.zeros_like(t_proto_ref)

    mt = mt_ref[...]
    s_tokt = jnp.dot(s_feat_ref[0], mt, preferred_element_type=jnp.float32)
    t_tokt = jnp.dot(t_feat_ref[0], mt, preferred_element_type=jnp.float32)
    s_tok_ref[0] = s_tokt.T
    t_tok_ref[0] = t_tokt.T

    def stats(cnt):
        cnt_ig = cnt[0:1, :]
        maxc = cnt_ig
        mode = jnp.full((1, _N), _IGNORE, jnp.int32)
        for v in range(_NUM_CLASSES):
            cv = cnt[v + 1:v + 2, :]
            upd = cv > maxc
            maxc = jnp.where(upd, cv, maxc)
            mode = jnp.where(upd, v, mode)
        nvalid = _PP - cnt_ig
        num = jnp.where(mode == _IGNORE, 0.0, maxc)
        purity = num / jnp.maximum(nvalid, 1.0)
        mask = (purity >= _PURITY_T) & (nvalid > 0)
        return mask, mode

    s_mask, s_mode = stats(scnt_ref[0])
    t_mask, t_mode = stats(tcnt_ref[0])
    conf_tok = cp_ref[0]
    t_mask = t_mask & (conf_tok >= _CONF_T)

    iota = jax.lax.broadcasted_iota(jnp.int32, (_NUM_CLASSES, _N), 0)
    s_any = jnp.max(s_mask.astype(jnp.int32), axis=1, keepdims=True) > 0
    t_any = jnp.max(t_mask.astype(jnp.int32), axis=1, keepdims=True) > 0

    def proto_update(mask, mode, tokt, proto_ref, gate):
        oh = ((iota == mode) & mask).astype(jnp.float32)
        sums = jax.lax.dot_general(oh, tokt, (((1,), (1,)), ((), ())),
                                   preferred_element_type=jnp.float32)
        counts = jnp.sum(oh, axis=1, keepdims=True)
        means = sums / jnp.maximum(counts, 1.0)
        upd = (counts > 0) & gate
        prev = proto_ref[...]
        newp = jnp.where(upd, _MOM * prev + (1.0 - _MOM) * means, prev)
        proto_ref[...] = newp
        return newp

    s_newp = proto_update(s_mask, s_mode, s_tokt, s_proto_ref, s_any)
    t_newp = proto_update(t_mask, t_mode, t_tokt, t_proto_ref, t_any & s_any)

    def dists(mask, mode, tokt, newp):
        sel = (iota == jnp.clip(mode, 0, _NUM_CLASSES - 1)).astype(jnp.float32)
        z = jnp.dot(newp, tokt, preferred_element_type=jnp.float32)
        xp = jnp.sum(sel * z, axis=0, keepdims=True)
        x2 = jnp.sum(tokt * tokt, axis=0, keepdims=True)
        p2 = jnp.sum(newp * newp, axis=1, keepdims=True)
        p2t = jnp.sum(sel * p2, axis=0, keepdims=True)
        d2 = jnp.maximum(x2 - 2.0 * xp + p2t, 0.0)
        return jnp.sqrt(d2) * mask.astype(jnp.float32)

    s_d_ref[0] = dists(s_mask, s_mode, s_tokt, s_newp)
    t_d_ref[0] = dists(t_mask, t_mode, t_tokt, t_newp)


def kernel(s_feat_map, t_feat_map, s_label_pixel, t_probs_pixel):
    cols = jnp.asarray(_COLS)
    rowsum = jnp.asarray(_RS)

    tcnt, scnt, csum = pl.pallas_call(
        _stats_body,
        grid=(_B, _RT),
        in_specs=[
            pl.BlockSpec((_IMG, _HF), lambda b, r: (0, 0)),
            pl.BlockSpec((_NBLK * _RT, _E_ROWS), lambda b, r: (0, 0)),
            pl.BlockSpec((1, _NUM_CLASSES, _ROWS, _IMG),
                         lambda b, r: (b, 0, r, 0)),
            pl.BlockSpec((1, _ROWS, _IMG), lambda b, r: (b, r, 0)),
        ],
        out_specs=[
            pl.BlockSpec((1, _NV, _RT, _HF), lambda b, r: (b, 0, r, 0)),
            pl.BlockSpec((1, _NV, _RT, _HF), lambda b, r: (b, 0, r, 0)),
            pl.BlockSpec((1, _RT, _HF), lambda b, r: (b, r, 0)),
        ],
        out_shape=[
            jax.ShapeDtypeStruct((_B, _NV, _HF, _HF), jnp.float32),
            jax.ShapeDtypeStruct((_B, _NV, _HF, _HF), jnp.float32),
            jax.ShapeDtypeStruct((_B, _HF, _HF), jnp.float32),
        ],
        compiler_params=pltpu.CompilerParams(
            dimension_semantics=("parallel", "arbitrary")),
    )(cols, rowsum, t_probs_pixel, s_label_pixel.astype(jnp.int32))

    scnt = scnt.reshape(_B, _NV, _N)
    tcnt = tcnt.reshape(_B, _NV, _N)
    csum = csum.reshape(_B, 1, _N)
    s_feat = s_feat_map.reshape(_B, _C, _HC * _HC)
    t_feat = t_feat_map.reshape(_B, _C, _HC * _HC)
    mt = jnp.asarray(_MT)

    s_tok, t_tok, s_proto, t_proto, s_d, t_d = pl.pallas_call(
        _main_body,
        grid=(_B,),
        in_specs=[
            pl.BlockSpec((_HC * _HC, _N), lambda b: (0, 0)),
            pl.BlockSpec((1, _C, _HC * _HC), lambda b: (b, 0, 0)),
            pl.BlockSpec((1, _C, _HC * _HC), lambda b: (b, 0, 0)),
            pl.BlockSpec((1, _NV, _N), lambda b: (b, 0, 0)),
            pl.BlockSpec((1, _NV, _N), lambda b: (b, 0, 0)),
            pl.BlockSpec((1, 1, _N), lambda b: (b, 0, 0)),
        ],
        out_specs=[
            pl.BlockSpec((1, _N, _C), lambda b: (b, 0, 0)),
            pl.BlockSpec((1, _N, _C), lambda b: (b, 0, 0)),
            pl.BlockSpec((_NUM_CLASSES, _C), lambda b: (0, 0)),
            pl.BlockSpec((_NUM_CLASSES, _C), lambda b: (0, 0)),
            pl.BlockSpec((1, 1, _N), lambda b: (b, 0, 0)),
            pl.BlockSpec((1, 1, _N), lambda b: (b, 0, 0)),
        ],
        out_shape=[
            jax.ShapeDtypeStruct((_B, _N, _C), jnp.float32),
            jax.ShapeDtypeStruct((_B, _N, _C), jnp.float32),
            jax.ShapeDtypeStruct((_NUM_CLASSES, _C), jnp.float32),
            jax.ShapeDtypeStruct((_NUM_CLASSES, _C), jnp.float32),
            jax.ShapeDtypeStruct((_B, 1, _N), jnp.float32),
            jax.ShapeDtypeStruct((_B, 1, _N), jnp.float32),
        ],
    )(mt, s_feat, t_feat, scnt, tcnt, csum)

    return (s_tok, t_tok, s_proto, t_proto,
            s_d.reshape(_B, _N), t_d.reshape(_B, _N))
